# baseline (speedup 1.0000x reference)
import functools

import jax
import jax.numpy as jnp
from jax import lax
from jax.experimental import pallas as pl
from jax.experimental.pallas import tpu as pltpu
from jax.experimental.pallas import tpu_sc as plsc

NC = 2
NS = 16
NW = NC * NS
LANES = 16
EPSILON = 1e-6
NACC = 4


def _rsqrt(x):
    i = lax.bitcast_convert_type(x, jnp.int32)
    i = jnp.int32(0x5F3759DF) - (i >> 1)
    y = lax.bitcast_convert_type(i, jnp.float32)
    for _ in range(3):
        y = y * (jnp.float32(1.5) - jnp.float32(0.5) * x * y * y)
    return y


def _tree_sum(vals):
    vals = list(vals)
    while len(vals) > 1:
        nxt = [a + b for a, b in zip(vals[0::2], vals[1::2])]
        if len(vals) % 2:
            nxt.append(vals[-1])
        vals = nxt
    return vals[0]


def kernel(input_ids, token_table, pos_table, ln_scale, ln_bias):
    B, S = input_ids.shape
    V, H = token_table.shape
    SP = S // NW
    NJ = H // LANES

    assert S % NW == 0 and H % LANES == 0 and SP == LANES and B % 2 == 0

    ids_w = (input_ids.astype(jnp.int32)
             .reshape(B, NW, SP).transpose(1, 0, 2).reshape(NW, B * SP))

    mesh = plsc.VectorSubcoreMesh(core_axis_name="c", subcore_axis_name="s")

    @functools.partial(
        pl.kernel,
        mesh=mesh,
        out_type=jax.ShapeDtypeStruct((B, S, H), jnp.float32),
        compiler_params=pltpu.CompilerParams(needs_layout_passes=False),
        scratch_types=[
            pltpu.VMEM((B * SP,), jnp.int32),
            pltpu.VMEM((SP, H), jnp.float32),
            pltpu.VMEM((H,), jnp.float32),
            pltpu.VMEM((H,), jnp.float32),
            pltpu.VMEM((SP, H), jnp.float32),
            pltpu.VMEM((SP, H), jnp.float32),
            pltpu.VMEM((SP, H), jnp.float32),
            pltpu.VMEM((SP, H), jnp.float32),
            pltpu.VMEM((SP, H), jnp.float32),
            pltpu.VMEM((SP, H), jnp.float32),
            pltpu.VMEM((SP, H), jnp.float32),
            pltpu.VMEM((SP, H), jnp.float32),
            pltpu.SMEM((2, LANES), jnp.float32),
            pltpu.SemaphoreType.DMA,
            pltpu.SemaphoreType.DMA,
            pltpu.SemaphoreType.DMA,
            pltpu.SemaphoreType.DMA,
            pltpu.SemaphoreType.DMA,
            pltpu.SemaphoreType.DMA,
            pltpu.SemaphoreType.DMA,
            pltpu.SemaphoreType.DMA,
        ],
    )
    def emb_kernel(ids_hbm, tok_hbm, pos_hbm, scale_hbm, bias_hbm, out_hbm,
                   idx_v, pos_v, scale_v, bias_v, in0, in1, in2, in3,
                   ou0, ou1, ou2, ou3, stat_v,
                   gi0, gi1, gi2, gi3, go0, go1, go2, go3):
        wid = lax.axis_index("s") * NC + lax.axis_index("c")
        s0 = wid * SP

        pltpu.sync_copy(ids_hbm.at[wid], idx_v)
        cp_pos = pltpu.async_copy(pos_hbm.at[pl.ds(s0, SP), :], pos_v, go0)
        cp_sc = pltpu.async_copy(scale_hbm, scale_v, go0)
        cp_bi = pltpu.async_copy(bias_hbm, bias_v, go0)

        inv_h = jnp.float32(1.0 / H)
        ins, outs = (in0, in1, in2, in3), (ou0, ou1, ou2, ou3)
        gis, gos = (gi0, gi1, gi2, gi3), (go0, go1, go2, go3)
        ND = 4

        def gather_start(b, buf, sem):
            pltpu.async_copy(tok_hbm.at[idx_v.at[pl.ds(b * SP, SP)]], buf, sem)

        def gather_wait(b, buf, sem):
            pltpu.make_async_copy(
                tok_hbm.at[idx_v.at[pl.ds(b * SP, SP)]], buf, sem).wait()

        def write_start(b, buf, sem):
            pltpu.async_copy(buf, out_hbm.at[b, pl.ds(s0, SP), :], sem)

        def write_wait(b, buf, sem):
            pltpu.make_async_copy(
                buf, out_hbm.at[b, pl.ds(s0, SP), :], sem).wait()

        def pass1(src, dst):
            @plsc.parallel_loop(0, SP)
            def one_row(r):
                accs = []
                accq = []
                for j in range(NJ):
                    sl = pl.ds(j * LANES, LANES)
                    x = src[r, sl] + pos_v[r, sl]
                    dst[r, sl] = x
                    if j < NACC:
                        accs.append(x)
                        accq.append(x * x)
                    else:
                        k = j % NACC
                        accs[k] = accs[k] + x
                        accq[k] = accq[k] + x * x
                mean = jnp.sum(_tree_sum(accs)) * inv_h
                var = jnp.sum(_tree_sum(accq)) * inv_h - mean * mean
                rstd = _rsqrt(var + jnp.float32(EPSILON))
                stat_v[0, r] = rstd
                stat_v[1, r] = -(mean * rstd)

        def pass2(dst):
            a_s = [stat_v[0, r] for r in range(SP)]
            b_s = [stat_v[1, r] for r in range(SP)]

            @plsc.parallel_loop(0, NJ)
            def colblk(j):
                sl = pl.ds(j * LANES, LANES)
                sc = scale_v[sl]
                bi = bias_v[sl]
                for r in range(SP):
                    x = dst[r, sl]
                    dst[r, sl] = (x * a_s[r] + b_s[r]) * sc + bi

        for p in range(ND):
            gather_start(p, ins[p], gis[p])
        cp_pos.wait()
        cp_sc.wait()
        cp_bi.wait()

        def group(i, carry):
            for p in range(ND):
                b = ND * i + p
                gather_wait(b, ins[p], gis[p])

                @pl.when(i >= 1)
                def _():
                    write_wait(b - ND, outs[p], gos[p])

                pass1(ins[p], outs[p])

                @pl.when(i < (B // ND - 1))
                def _():
                    gather_start(b + ND, ins[p], gis[p])

                pass2(outs[p])
                write_start(b, outs[p], gos[p])
            return carry

        lax.fori_loop(0, B // ND, group, 0)
        for p in range(ND):
            write_wait(B - ND + p, outs[p], gos[p])

    return emb_kernel(ids_w, token_table, pos_table, ln_scale, ln_bias)


# speedup vs baseline: 1.3951x; 1.3951x over previous
import functools

import jax
import jax.numpy as jnp
from jax import lax
from jax.experimental import pallas as pl
from jax.experimental.pallas import tpu as pltpu
from jax.experimental.pallas import tpu_sc as plsc

NC = 2
NS = 16
NW = NC * NS
LANES = 16
EPSILON = 1e-6
NACC = 4


def _rsqrt(x):
    i = lax.bitcast_convert_type(x, jnp.int32)
    i = jnp.int32(0x5F3759DF) - (i >> 1)
    y = lax.bitcast_convert_type(i, jnp.float32)
    for _ in range(3):
        y = y * (jnp.float32(1.5) - jnp.float32(0.5) * x * y * y)
    return y


def _tree_sum(vals):
    vals = list(vals)
    while len(vals) > 1:
        nxt = [a + b for a, b in zip(vals[0::2], vals[1::2])]
        if len(vals) % 2:
            nxt.append(vals[-1])
        vals = nxt
    return vals[0]


def kernel(input_ids, token_table, pos_table, ln_scale, ln_bias):
    B, S = input_ids.shape
    V, H = token_table.shape
    SP = S // NW
    NJ = H // LANES

    assert S % NW == 0 and H % LANES == 0 and SP == LANES and B % 2 == 0

    ids_w = (input_ids.astype(jnp.int32)
             .reshape(B, NW, SP).transpose(1, 0, 2).reshape(NW, B * SP))

    mesh = plsc.VectorSubcoreMesh(core_axis_name="c", subcore_axis_name="s")

    @functools.partial(
        pl.kernel,
        mesh=mesh,
        out_type=jax.ShapeDtypeStruct((B, S, H), jnp.float32),
        compiler_params=pltpu.CompilerParams(needs_layout_passes=False),
        scratch_types=[
            pltpu.VMEM((B * SP,), jnp.int32),
            pltpu.VMEM((SP, H), jnp.float32),
            pltpu.VMEM((H,), jnp.float32),
            pltpu.VMEM((H,), jnp.float32),
            pltpu.VMEM((SP, H), jnp.float32),
            pltpu.VMEM((SP, H), jnp.float32),
            pltpu.VMEM((SP, H), jnp.float32),
            pltpu.VMEM((SP, H), jnp.float32),
            pltpu.SMEM((2, LANES), jnp.float32),
            pltpu.SemaphoreType.DMA,
            pltpu.SemaphoreType.DMA,
            pltpu.SemaphoreType.DMA,
            pltpu.SemaphoreType.DMA,
        ],
    )
    def emb_kernel(ids_hbm, tok_hbm, pos_hbm, scale_hbm, bias_hbm, out_hbm,
                   idx_v, pos_v, scale_v, bias_v, in0, in1, ou0, ou1, stat_v,
                   gi0, gi1, go0, go1):
        wid = lax.axis_index("s") * NC + lax.axis_index("c")
        s0 = wid * SP

        pltpu.sync_copy(ids_hbm.at[wid], idx_v)
        cp_pos = pltpu.async_copy(pos_hbm.at[pl.ds(s0, SP), :], pos_v, go0)
        cp_sc = pltpu.async_copy(scale_hbm, scale_v, go0)
        cp_bi = pltpu.async_copy(bias_hbm, bias_v, go0)

        inv_h = jnp.float32(1.0 / H)
        ins, outs = (in0, in1), (ou0, ou1)
        gis, gos = (gi0, gi1), (go0, go1)
        ND = 2

        def gather_start(b, buf, sem):
            pltpu.async_copy(tok_hbm.at[idx_v.at[pl.ds(b * SP, SP)]], buf, sem)

        def gather_wait(b, buf, sem):
            pltpu.make_async_copy(
                tok_hbm.at[idx_v.at[pl.ds(b * SP, SP)]], buf, sem).wait()

        def write_start(b, buf, sem):
            pltpu.async_copy(buf, out_hbm.at[b, pl.ds(s0, SP), :], sem)

        def write_wait(b, buf, sem):
            pltpu.make_async_copy(
                buf, out_hbm.at[b, pl.ds(s0, SP), :], sem).wait()

        def pass1(src, dst):
            @plsc.parallel_loop(0, SP)
            def one_row(r):
                accs = []
                accq = []
                for j in range(NJ):
                    sl = pl.ds(j * LANES, LANES)
                    x = src[r, sl] + pos_v[r, sl]
                    dst[r, sl] = x
                    if j < NACC:
                        accs.append(x)
                        accq.append(x * x)
                    else:
                        k = j % NACC
                        accs[k] = accs[k] + x
                        accq[k] = accq[k] + x * x
                mean = jnp.sum(_tree_sum(accs)) * inv_h
                var = jnp.sum(_tree_sum(accq)) * inv_h - mean * mean
                rstd = _rsqrt(var + jnp.float32(EPSILON))
                stat_v[0, r] = rstd
                stat_v[1, r] = -(mean * rstd)

        def pass2(dst):
            a_s = [stat_v[0, r] for r in range(SP)]
            b_s = [stat_v[1, r] for r in range(SP)]

            @plsc.parallel_loop(0, NJ)
            def colblk(j):
                sl = pl.ds(j * LANES, LANES)
                sc = scale_v[sl]
                bi = bias_v[sl]
                for r in range(SP):
                    x = dst[r, sl]
                    dst[r, sl] = (x * a_s[r] + b_s[r]) * sc + bi

        for p in range(ND):
            gather_start(p, ins[p], gis[p])
        cp_pos.wait()
        cp_sc.wait()
        cp_bi.wait()

        def group(i, carry):
            for p in range(ND):
                b = ND * i + p
                gather_wait(b, ins[p], gis[p])

                @pl.when(i >= 1)
                def _():
                    write_wait(b - ND, outs[p], gos[p])

                pass1(ins[p], outs[p])

                @pl.when(i < (B // ND - 1))
                def _():
                    gather_start(b + ND, ins[p], gis[p])

                pass2(outs[p])
                write_start(b, outs[p], gos[p])
            return carry

        lax.fori_loop(0, B // ND, group, 0)
        for p in range(ND):
            write_wait(B - ND + p, outs[p], gos[p])

    return emb_kernel(ids_w, token_table, pos_table, ln_scale, ln_bias)
